# baseline (speedup 1.0000x reference)
import functools

import jax
import jax.numpy as jnp
from jax import lax
from jax.experimental import pallas as pl
from jax.experimental.pallas import tpu as pltpu
from jax.experimental.pallas import tpu_sc as plsc

_NC = 2
_NS = 16
_NW = _NC * _NS
_CHUNK = 128
_NBUF = 3


@functools.lru_cache(maxsize=None)
def _make_gather(total, d, b, s):
    n = b * s
    assert n % (_NW * _CHUNK) == 0
    per_w = n // _NW
    nchunk = per_w // _CHUNK
    mesh = plsc.VectorSubcoreMesh(core_axis_name="c", subcore_axis_name="s")

    @functools.partial(
        pl.kernel,
        mesh=mesh,
        out_type=jax.ShapeDtypeStruct((n, d), jnp.float32),
        scratch_types=[
            pltpu.VMEM((per_w,), jnp.int32),
        ]
        + [pltpu.VMEM((_CHUNK, d), jnp.float32) for _ in range(_NBUF)]
        + [
            pltpu.SemaphoreType.DMA,
            pltpu.SemaphoreType.DMA,
        ],
    )
    def k(idx_hbm, table_hbm, out_hbm, idx_v, *rest):
        bufs, (gsem, ssem) = rest[:_NBUF], rest[_NBUF:]
        wid = lax.axis_index("s") * _NC + lax.axis_index("c")
        pltpu.sync_copy(idx_hbm.at[wid], idx_v)
        base = wid * per_w
        gathers, stores = [], []

        def gather(j):
            gathers.append(pltpu.async_copy(
                table_hbm.at[idx_v.at[pl.ds(j * _CHUNK, _CHUNK)]],
                bufs[j % _NBUF], gsem))

        def store(j):
            stores.append(pltpu.async_copy(
                bufs[j % _NBUF],
                out_hbm.at[pl.ds(base + j * _CHUNK, _CHUNK)], ssem))

        gather(0)
        for j in range(nchunk):
            if j + 1 < nchunk:
                if j - (_NBUF - 1) >= 0:
                    stores[j - (_NBUF - 1)].wait()
                gather(j + 1)
            gathers[j].wait()
            store(j)
        for j in range(max(0, nchunk - (_NBUF - 1) - 1), nchunk):
            stores[j].wait()

    return k


def kernel(x, table):
    b, s = x.shape
    total, d = table.shape
    out = _make_gather(total, d, b, s)(x.reshape(_NW, -1), table)
    return out.reshape(b, s, d)


# speedup vs baseline: 1.0130x; 1.0130x over previous
import functools

import jax
import jax.numpy as jnp
from jax import lax
from jax.experimental import pallas as pl
from jax.experimental.pallas import tpu as pltpu
from jax.experimental.pallas import tpu_sc as plsc

_NC = 2
_NS = 16
_NW = _NC * _NS
_CHUNK = 128


@functools.lru_cache(maxsize=None)
def _make_gather(total, d, b, s):
    n = b * s
    assert n % (_NW * 2 * _CHUNK) == 0
    per_w = n // _NW
    npair = per_w // (2 * _CHUNK)
    mesh = plsc.VectorSubcoreMesh(core_axis_name="c", subcore_axis_name="s")

    @functools.partial(
        pl.kernel,
        mesh=mesh,
        out_type=jax.ShapeDtypeStruct((n, d), jnp.float32),
        scratch_types=[
            pltpu.VMEM((per_w,), jnp.int32),
            pltpu.VMEM((_CHUNK, d), jnp.float32),
            pltpu.VMEM((_CHUNK, d), jnp.float32),
            pltpu.SemaphoreType.DMA,
            pltpu.SemaphoreType.DMA,
        ],
    )
    def k(idx_hbm, table_hbm, out_hbm, idx_v, buf0, buf1, sem0, sem1):
        wid = lax.axis_index("s") * _NC + lax.axis_index("c")
        pltpu.sync_copy(idx_hbm.at[wid], idx_v)
        base = wid * per_w

        def gather(off, buf, sem):
            pltpu.async_copy(table_hbm.at[idx_v.at[pl.ds(off, _CHUNK)]],
                             buf, sem)

        def wait_gather(buf, sem):
            pltpu.make_async_copy(table_hbm.at[idx_v.at[pl.ds(0, _CHUNK)]],
                                  buf, sem).wait()

        def store(buf, off):
            pltpu.sync_copy(
                buf, out_hbm.at[pl.ds(pl.multiple_of(base + off, _CHUNK),
                                      _CHUNK)])

        gather(0, buf0, sem0)

        def body(i, _):
            c0 = pl.multiple_of(2 * _CHUNK * i, 2 * _CHUNK)
            gather(c0 + _CHUNK, buf1, sem1)
            wait_gather(buf0, sem0)
            store(buf0, c0)

            @pl.when(i < npair - 1)
            def _():
                gather(c0 + 2 * _CHUNK, buf0, sem0)

            wait_gather(buf1, sem1)
            store(buf1, c0 + _CHUNK)
            return _

        lax.fori_loop(0, npair, body, None)

    return k


def kernel(x, table):
    b, s = x.shape
    total, d = table.shape
    out = _make_gather(total, d, b, s)(x.reshape(_NW, -1), table)
    return out.reshape(b, s, d)


# speedup vs baseline: 1.0229x; 1.0097x over previous
import functools

import jax
import jax.numpy as jnp
from jax import lax
from jax.experimental import pallas as pl
from jax.experimental.pallas import tpu as pltpu
from jax.experimental.pallas import tpu_sc as plsc

_NC = 2
_NS = 16
_NW = _NC * _NS
_CHUNK = 128


@functools.lru_cache(maxsize=None)
def _make_gather(total, d, b, s):
    n = b * s
    assert n % (_NW * 2 * _CHUNK) == 0
    per_w = n // _NW
    npair = per_w // (2 * _CHUNK)
    mesh = plsc.VectorSubcoreMesh(core_axis_name="c", subcore_axis_name="s")

    @functools.partial(
        pl.kernel,
        mesh=mesh,
        out_type=jax.ShapeDtypeStruct((n, d), jnp.float32),
        scratch_types=[
            pltpu.VMEM((per_w,), jnp.int32),
            pltpu.VMEM((_CHUNK, d), jnp.float32),
            pltpu.VMEM((_CHUNK, d), jnp.float32),
            pltpu.SemaphoreType.DMA,
            pltpu.SemaphoreType.DMA,
        ],
    )
    def k(idx_hbm, table_hbm, out_hbm, idx_v, buf0, buf1, sem0, sem1):
        wid = lax.axis_index("s") * _NC + lax.axis_index("c")
        pltpu.async_copy(idx_hbm.at[wid, pl.ds(0, _CHUNK)],
                         idx_v.at[pl.ds(0, _CHUNK)], sem0).wait()
        base = wid * per_w

        def gather(off, buf, sem):
            pltpu.async_copy(table_hbm.at[idx_v.at[pl.ds(off, _CHUNK)]],
                             buf, sem)

        def wait_gather(buf, sem):
            pltpu.make_async_copy(table_hbm.at[idx_v.at[pl.ds(0, _CHUNK)]],
                                  buf, sem).wait()

        def store(buf, off):
            pltpu.sync_copy(
                buf, out_hbm.at[pl.ds(pl.multiple_of(base + off, _CHUNK),
                                      _CHUNK)])

        gather(0, buf0, sem0)
        pltpu.sync_copy(idx_hbm.at[wid, pl.ds(_CHUNK, per_w - _CHUNK)],
                        idx_v.at[pl.ds(_CHUNK, per_w - _CHUNK)])

        def body(i, _):
            c0 = pl.multiple_of(2 * _CHUNK * i, 2 * _CHUNK)
            gather(c0 + _CHUNK, buf1, sem1)
            wait_gather(buf0, sem0)
            store(buf0, c0)

            @pl.when(i < npair - 1)
            def _():
                gather(c0 + 2 * _CHUNK, buf0, sem0)

            wait_gather(buf1, sem1)
            store(buf1, c0 + _CHUNK)
            return _

        lax.fori_loop(0, npair, body, None)

    return k


def kernel(x, table):
    b, s = x.shape
    total, d = table.shape
    out = _make_gather(total, d, b, s)(x.reshape(_NW, -1), table)
    return out.reshape(b, s, d)
